# baseline (speedup 1.0000x reference)
import functools
import math

import jax
import jax.numpy as jnp
from jax import lax
from jax.experimental import pallas as pl
from jax.experimental.pallas import tpu as pltpu
from jax.experimental.pallas import tpu_sc as plsc

D_MODEL = 64
SCALE = math.sqrt(D_MODEL)
NUM_WORKERS = 32
SEQ = 200
BATCH = 4096
BLK = 128
LANES = 16
VOCAB = 1000000
ROWPAD = 128
CHUNK_ROWS = 8000
N_CHUNKS = VOCAB // CHUNK_ROWS
TPITCH = 137


@functools.partial(
    pl.kernel,
    mesh=plsc.VectorSubcoreMesh(core_axis_name="c", subcore_axis_name="s"),
    out_type=jax.ShapeDtypeStruct((SEQ, 8, NUM_WORKERS, 8, 128), jnp.float32),
    scratch_types=[
        pltpu.VMEM((SEQ, BLK), jnp.int32),
        pltpu.VMEM((BLK, D_MODEL), jnp.float32),
        pltpu.VMEM((BLK, D_MODEL), jnp.float32),
        pltpu.VMEM((D_MODEL, TPITCH), jnp.float32),
        pltpu.VMEM((D_MODEL, TPITCH), jnp.float32),
        pltpu.SemaphoreType.DMA,
        pltpu.SemaphoreType.DMA,
        pltpu.SemaphoreType.DMA,
        pltpu.SemaphoreType.DMA,
    ],
    compiler_params=pltpu.CompilerParams(
        use_tc_tiling_on_sc=False, needs_layout_passes=False
    ),
)
def _emb_lookup(
    xt_hbm, tab_hbm, out_hbm, idx_v, rows0, rows1, tb0, tb1, g0, g1, t0, t1
):
    w = lax.axis_index("s") * 2 + lax.axis_index("c")

    pltpu.sync_copy(xt_hbm.at[:, pl.ds(w * BLK, BLK)], idx_v)

    iota = lax.iota(jnp.int32, LANES)
    d_sel = [dg * LANES + iota for dg in range(D_MODEL // LANES)]

    def start_gather(s, buf, gsem):
        pltpu.make_async_copy(tab_hbm.at[idx_v.at[s]], buf, gsem).start()

    def wait_gather(buf, gsem):
        pltpu.make_async_copy(tab_hbm.at[idx_v.at[0]], buf, gsem).wait()

    def start_write(s, tbuf, tsem):
        for dt in range(8):
            pltpu.make_async_copy(
                tbuf.at[pl.ds(dt * 8, 8), pl.ds(0, 128)],
                out_hbm.at[s, dt, w],
                tsem,
            ).start()

    def wait_write(tbuf, tsem):
        for dt in range(8):
            pltpu.make_async_copy(
                tbuf.at[pl.ds(0, 8), pl.ds(0, 128)],
                out_hbm.at[0, dt, w],
                tsem,
            ).wait()

    def transpose_scale(rows, tbuf):
        @plsc.parallel_loop(0, BLK, unroll=4)
        def _(b):
            bvec = jnp.full((LANES,), b, jnp.int32)
            for dg in range(D_MODEL // LANES):
                v = rows[b, pl.ds(dg * LANES, LANES)]
                plsc.store_scatter(tbuf, [d_sel[dg], bvec], v * SCALE)

    start_gather(0, rows0, g0)
    start_gather(1, rows1, g1)

    def phase(s, rows, tbuf, gsem, tsem, first, last):
        wait_gather(rows, gsem)
        if not first:
            wait_write(tbuf, tsem)
        transpose_scale(rows, tbuf)
        if not last:
            start_gather(s + 2, rows, gsem)
        start_write(s, tbuf, tsem)

    phase(0, rows0, tb0, g0, t0, True, False)
    phase(1, rows1, tb1, g1, t1, True, False)

    def pair_body(k, _):
        s0 = 2 * k
        phase(s0, rows0, tb0, g0, t0, False, False)
        phase(s0 + 1, rows1, tb1, g1, t1, False, False)
        return 0

    lax.fori_loop(1, SEQ // 2 - 1, pair_body, 0)

    phase(SEQ - 2, rows0, tb0, g0, t0, False, True)
    phase(SEQ - 1, rows1, tb1, g1, t1, False, True)
    wait_write(tb0, t0)
    wait_write(tb1, t1)


def kernel(x, lut):
    xt = jnp.transpose(x).astype(jnp.int32)
    out_phys = _emb_lookup(xt, lut)
    return jnp.transpose(out_phys, (2, 4, 0, 1, 3)).reshape(BATCH, SEQ, D_MODEL)


# speedup vs baseline: 1.0006x; 1.0006x over previous
import functools
import math

import jax
import jax.numpy as jnp
from jax import lax
from jax.experimental import pallas as pl
from jax.experimental.pallas import tpu as pltpu
from jax.experimental.pallas import tpu_sc as plsc

D_MODEL = 64
SCALE = math.sqrt(D_MODEL)
NUM_WORKERS = 32
SEQ = 200
BATCH = 4096
BLK = 128
LANES = 16
VOCAB = 1000000
ROWPAD = 128
CHUNK_ROWS = 8000
N_CHUNKS = VOCAB // CHUNK_ROWS
TPITCH = 137


@functools.partial(
    pl.kernel,
    mesh=plsc.VectorSubcoreMesh(core_axis_name="c", subcore_axis_name="s"),
    out_type=jax.ShapeDtypeStruct((SEQ, 8, NUM_WORKERS, 8, 128), jnp.float32),
    scratch_types=[
        pltpu.VMEM((SEQ, BLK), jnp.int32),
        pltpu.VMEM((BLK, ROWPAD), jnp.float32),
        pltpu.VMEM((BLK, ROWPAD), jnp.float32),
        pltpu.VMEM((D_MODEL, TPITCH), jnp.float32),
        pltpu.VMEM((D_MODEL, TPITCH), jnp.float32),
        pltpu.SemaphoreType.DMA,
        pltpu.SemaphoreType.DMA,
        pltpu.SemaphoreType.DMA,
        pltpu.SemaphoreType.DMA,
    ],
    compiler_params=pltpu.CompilerParams(
        use_tc_tiling_on_sc=False, needs_layout_passes=False
    ),
)
def _emb_lookup(
    xt_hbm, tab_hbm, out_hbm, idx_v, rows0, rows1, tb0, tb1, g0, g1, t0, t1
):
    w = lax.axis_index("s") * 2 + lax.axis_index("c")

    pltpu.sync_copy(xt_hbm.at[:, pl.ds(w * BLK, BLK)], idx_v)

    iota = lax.iota(jnp.int32, LANES)
    d_sel = [dg * LANES + iota for dg in range(D_MODEL // LANES)]

    def start_gather(s, buf, gsem):
        pltpu.make_async_copy(tab_hbm.at[idx_v.at[s]], buf, gsem).start()

    def wait_gather(buf, gsem):
        pltpu.make_async_copy(tab_hbm.at[idx_v.at[0]], buf, gsem).wait()

    def start_write(s, tbuf, tsem):
        for dt in range(8):
            pltpu.make_async_copy(
                tbuf.at[pl.ds(dt * 8, 8), pl.ds(0, 128)],
                out_hbm.at[s, dt, w],
                tsem,
            ).start()

    def wait_write(tbuf, tsem):
        for dt in range(8):
            pltpu.make_async_copy(
                tbuf.at[pl.ds(0, 8), pl.ds(0, 128)],
                out_hbm.at[0, dt, w],
                tsem,
            ).wait()

    def transpose_scale(rows, tbuf):
        @plsc.parallel_loop(0, BLK, unroll=4)
        def _(b):
            bvec = jnp.full((LANES,), b, jnp.int32)
            for dg in range(D_MODEL // LANES):
                v = rows[b, pl.ds(dg * LANES, LANES)]
                plsc.store_scatter(tbuf, [d_sel[dg], bvec], v * SCALE)

    start_gather(0, rows0, g0)
    start_gather(1, rows1, g1)

    def phase(s, rows, tbuf, gsem, tsem, first, last):
        wait_gather(rows, gsem)
        if not first:
            wait_write(tbuf, tsem)
        transpose_scale(rows, tbuf)
        if not last:
            start_gather(s + 2, rows, gsem)
        start_write(s, tbuf, tsem)

    phase(0, rows0, tb0, g0, t0, True, False)
    phase(1, rows1, tb1, g1, t1, True, False)

    def pair_body(k, _):
        s0 = 2 * k
        phase(s0, rows0, tb0, g0, t0, False, False)
        phase(s0 + 1, rows1, tb1, g1, t1, False, False)
        return 0

    lax.fori_loop(1, SEQ // 2 - 1, pair_body, 0)

    phase(SEQ - 2, rows0, tb0, g0, t0, False, True)
    phase(SEQ - 1, rows1, tb1, g1, t1, False, True)
    wait_write(tb0, t0)
    wait_write(tb1, t1)


def kernel(x, lut):
    xt = jnp.transpose(x).astype(jnp.int32)
    tab = jnp.pad(lut, ((0, 0), (0, ROWPAD - D_MODEL)))
    out_phys = _emb_lookup(xt, tab)
    return jnp.transpose(out_phys, (2, 4, 0, 1, 3)).reshape(BATCH, SEQ, D_MODEL)


# speedup vs baseline: 1.0874x; 1.0867x over previous
import functools
import math

import jax
import jax.numpy as jnp
from jax import lax
from jax.experimental import pallas as pl
from jax.experimental.pallas import tpu as pltpu
from jax.experimental.pallas import tpu_sc as plsc

D_MODEL = 64
SCALE = math.sqrt(D_MODEL)
NUM_WORKERS = 32
SEQ = 200
BATCH = 4096
BLK = 128
LANES = 16
VOCAB = 1000000
ROWPAD = 128
CHUNK_ROWS = 8000
N_CHUNKS = VOCAB // CHUNK_ROWS
TPITCH = 137


@functools.partial(
    pl.kernel,
    mesh=plsc.VectorSubcoreMesh(core_axis_name="c", subcore_axis_name="s"),
    out_type=jax.ShapeDtypeStruct((SEQ, 8, NUM_WORKERS, 8, 128), jnp.float32),
    scratch_types=[
        pltpu.VMEM((SEQ, BLK), jnp.int32),
        pltpu.VMEM((BLK, D_MODEL), jnp.float32),
        pltpu.VMEM((BLK, D_MODEL), jnp.float32),
        pltpu.VMEM((D_MODEL, TPITCH), jnp.float32),
        pltpu.VMEM((D_MODEL, TPITCH), jnp.float32),
        pltpu.SemaphoreType.DMA,
        pltpu.SemaphoreType.DMA,
        pltpu.SemaphoreType.DMA,
        pltpu.SemaphoreType.DMA,
    ],
    compiler_params=pltpu.CompilerParams(
        use_tc_tiling_on_sc=False, needs_layout_passes=False
    ),
)
def _emb_lookup(
    xt_hbm, tab_hbm, out_hbm, idx_v, rows0, rows1, tb0, tb1, g0, g1, t0, t1
):
    w = lax.axis_index("s") * 2 + lax.axis_index("c")

    pltpu.sync_copy(xt_hbm.at[:, pl.ds(w * BLK, BLK)], idx_v)

    @plsc.parallel_loop(0, SEQ, unroll=4)
    def _(s):
        for lg in range(BLK // LANES):
            sl = pl.ds(lg * LANES, LANES)
            idx_v[s, sl] = idx_v[s, sl] * 2

    iota = lax.iota(jnp.int32, LANES)
    d_sel = [dg * LANES + iota for dg in range(D_MODEL // LANES)]

    def start_gather(s, buf, gsem):
        pltpu.make_async_copy(tab_hbm.at[idx_v.at[s]], buf, gsem).start()

    def wait_gather(buf, gsem):
        pltpu.make_async_copy(tab_hbm.at[idx_v.at[0]], buf, gsem).wait()

    def start_write(s, tbuf, tsem):
        for dt in range(8):
            pltpu.make_async_copy(
                tbuf.at[pl.ds(dt * 8, 8), pl.ds(0, 128)],
                out_hbm.at[s, dt, w],
                tsem,
            ).start()

    def wait_write(tbuf, tsem):
        for dt in range(8):
            pltpu.make_async_copy(
                tbuf.at[pl.ds(0, 8), pl.ds(0, 128)],
                out_hbm.at[0, dt, w],
                tsem,
            ).wait()

    def transpose_scale(rows, tbuf):
        @plsc.parallel_loop(0, BLK, unroll=4)
        def _(b):
            bvec = jnp.full((LANES,), b, jnp.int32)
            for dg in range(D_MODEL // LANES):
                v = rows[b, pl.ds(dg * LANES, LANES)]
                plsc.store_scatter(tbuf, [d_sel[dg], bvec], v * SCALE)

    start_gather(0, rows0, g0)
    start_gather(1, rows1, g1)

    def phase(s, rows, tbuf, gsem, tsem, first, last):
        wait_gather(rows, gsem)
        if not first:
            wait_write(tbuf, tsem)
        transpose_scale(rows, tbuf)
        if not last:
            start_gather(s + 2, rows, gsem)
        start_write(s, tbuf, tsem)

    phase(0, rows0, tb0, g0, t0, True, False)
    phase(1, rows1, tb1, g1, t1, True, False)

    def pair_body(k, _):
        s0 = 2 * k
        phase(s0, rows0, tb0, g0, t0, False, False)
        phase(s0 + 1, rows1, tb1, g1, t1, False, False)
        return 0

    lax.fori_loop(1, SEQ // 2 - 1, pair_body, 0)

    phase(SEQ - 2, rows0, tb0, g0, t0, False, True)
    phase(SEQ - 1, rows1, tb1, g1, t1, False, True)
    wait_write(tb0, t0)
    wait_write(tb1, t1)


def kernel(x, lut):
    xt = jnp.transpose(x).astype(jnp.int32)
    tab = jnp.pad(lut, ((0, 0), (0, ROWPAD - D_MODEL)))
    out_phys = _emb_lookup(xt, tab.reshape(2 * VOCAB, D_MODEL))
    return jnp.transpose(out_phys, (2, 4, 0, 1, 3)).reshape(BATCH, SEQ, D_MODEL)
